# Initial kernel scaffold; baseline (speedup 1.0000x reference)
import jax
import jax.numpy as jnp
from jax.experimental import pallas as pl


def kernel(input_path, observe_path, theta, sigma_mat):
    raise NotImplementedError("write your pallas kernel here")


# speedup vs baseline: 2.8810x; 2.8810x over previous
import jax
import jax.numpy as jnp
from jax.experimental import pallas as pl

L_EXP = 5
T_STEPS = 20
N_PART = 2048
DX = 8
DY = 8
INIT_VAL = 0.0
L_SEG = 2 ** L_EXP


def _sim_segment_kernel(x_ref, sdw_ref, theta_ref, seg_ref):
    hl = jnp.float32(2.0 ** (-L_EXP))
    theta = theta_ref[...]

    def body(l, x):
        xn = x + (-theta * x) * hl + sdw_ref[l]
        seg_ref[l] = xn
        return xn

    jax.lax.fori_loop(0, L_SEG, body, x_ref[...])


def _simulate_segment(x_t, sdw_t, theta_col):
    return pl.pallas_call(
        _sim_segment_kernel,
        out_shape=jax.ShapeDtypeStruct((L_SEG, DX, N_PART), jnp.float32),
    )(x_t, sdw_t, theta_col)


def kernel(input_path, observe_path, theta, sigma_mat):
    key = jax.random.key(42)
    L = L_SEG
    hl = jnp.float32(2.0 ** (-L_EXP))
    theta_col = theta.reshape(DX, 1)

    def seg_noise(t):
        keys = jax.random.split(jax.random.fold_in(key, 2 * t), L)

        def one(k):
            dw = jax.random.normal(k, (N_PART, DX, 1), dtype=jnp.float32) * jnp.sqrt(hl)
            return (sigma_mat @ dw)[..., 0]

        return jax.lax.map(one, keys)

    sdw_all = jax.lax.map(seg_noise, jnp.arange(T_STEPS))
    sdw_all_t = sdw_all.transpose(0, 1, 3, 2)

    def dice_t(t):
        return jax.random.uniform(jax.random.fold_in(key, 2 * t + 1), (N_PART,), dtype=jnp.float32)

    dice_all = jax.lax.map(dice_t, jnp.arange(T_STEPS))

    x = jnp.full((DX, N_PART), INIT_VAL, dtype=jnp.float32)
    gn = jnp.zeros(N_PART, dtype=jnp.float32)

    def step(carry, t):
        x, gn = carry
        seg = _simulate_segment(x, sdw_all_t[t], theta_col)
        ui = L * (t + 1)
        x_end = seg[L - 1].T
        x_end = x_end.at[-1].set(input_path[ui])
        gn = -0.5 * jnp.sum((observe_path[t + 1] - x_end) ** 2, axis=-1) + gn
        what = jnp.exp(gn - jnp.max(gn))
        wn = what / jnp.sum(what)
        wn_d = jax.lax.stop_gradient(wn)
        ess = 1.0 / jnp.sum(wn_d ** 2)
        bins = jnp.cumsum(wn_d)
        bins = bins.at[-1].set(jnp.maximum(1.0, bins[-1]))
        idx = jnp.clip(jnp.digitize(dice_all[t], bins), 0, N_PART - 1)
        do = ess <= N_PART / 2.0
        x_new = jnp.where(do, x_end[idx], x_end)
        gn = jnp.where(do, jnp.zeros(N_PART, dtype=jnp.float32), gn)
        x_new = x_new.at[-1].set(input_path[ui])
        return (x_new.T, gn), (seg, idx, do, wn)

    (_, _), (segs, idx_all, do_all, wn_all) = jax.lax.scan(
        step, (x, gn), jnp.arange(T_STEPS))
    wn_final = wn_all[-1]

    dice1 = jax.random.uniform(jax.random.fold_in(key, 10 ** 6), (1,), dtype=jnp.float32)
    binsf = jnp.cumsum(jax.lax.stop_gradient(wn_final))
    binsf = binsf.at[-1].set(jnp.maximum(1.0, binsf[-1]))
    j = jnp.clip(jnp.digitize(dice1, binsf), 0, N_PART - 1)[0]

    def back(c, t):
        c_prev = jnp.where(do_all[t - 1], idx_all[t - 1, c], c)
        c_prev = jnp.where(c == N_PART - 1, N_PART - 1, c_prev)
        return c_prev, c

    c0, cs_rev = jax.lax.scan(back, j, jnp.arange(T_STEPS - 1, 0, -1))
    cs = jnp.concatenate([jnp.array([c0]), cs_rev[::-1]])

    def seg_rows(t):
        c = cs[t]
        own = segs[t, :, :, c]
        pin = jax.lax.dynamic_slice_in_dim(input_path, L * t + 1, L)
        return jnp.where(c == N_PART - 1, pin, own)

    rows = jax.vmap(seg_rows)(jnp.arange(T_STEPS)).reshape(T_STEPS * L, DX)
    row0 = jnp.where(cs[0] == N_PART - 1, input_path[0],
                     jnp.zeros(DX, jnp.float32))
    return jnp.concatenate([row0[None], rows], axis=0)


# speedup vs baseline: 3.7624x; 1.3059x over previous
import jax
import jax.numpy as jnp
from jax.experimental import pallas as pl

L_EXP = 5
T_STEPS = 20
N_PART = 2048
DX = 8
DY = 8
INIT_VAL = 0.0
L_SEG = 2 ** L_EXP


def _sim_segment_kernel(x_ref, sdw_ref, theta_ref, seg_ref):
    hl = jnp.float32(2.0 ** (-L_EXP))
    theta = theta_ref[...]

    def body(l, x):
        xn = x + (-theta * x) * hl + sdw_ref[l]
        seg_ref[l] = xn
        return xn

    jax.lax.fori_loop(0, L_SEG, body, x_ref[...])


def _simulate_segment(x_t, sdw_t, theta_col):
    return pl.pallas_call(
        _sim_segment_kernel,
        out_shape=jax.ShapeDtypeStruct((L_SEG, DX, N_PART), jnp.float32),
    )(x_t, sdw_t, theta_col)


def kernel(input_path, observe_path, theta, sigma_mat):
    key = jax.random.key(42)
    L = L_SEG
    hl = jnp.float32(2.0 ** (-L_EXP))
    theta_col = theta.reshape(DX, 1)

    step_keys = jax.vmap(lambda t: jax.random.fold_in(key, 2 * t))(jnp.arange(T_STEPS))
    sub_keys = jax.vmap(lambda k: jax.random.split(k, L))(step_keys)
    flat_keys = sub_keys.reshape(T_STEPS * L)

    dw_all = jax.vmap(
        lambda k: jax.random.normal(k, (N_PART, DX, 1), dtype=jnp.float32)
    )(flat_keys) * jnp.sqrt(hl)
    sdw_all = (sigma_mat @ dw_all)[..., 0].reshape(T_STEPS, L, N_PART, DX)
    sdw_all_t = sdw_all.transpose(0, 1, 3, 2)

    dice_all = jax.vmap(
        lambda t: jax.random.uniform(jax.random.fold_in(key, 2 * t + 1),
                                     (N_PART,), dtype=jnp.float32)
    )(jnp.arange(T_STEPS))

    x = jnp.full((DX, N_PART), INIT_VAL, dtype=jnp.float32)
    gn = jnp.zeros(N_PART, dtype=jnp.float32)

    def step(carry, t):
        x, gn = carry
        seg = _simulate_segment(x, sdw_all_t[t], theta_col)
        ui = L * (t + 1)
        x_end = seg[L - 1].T
        x_end = x_end.at[-1].set(input_path[ui])
        gn = -0.5 * jnp.sum((observe_path[t + 1] - x_end) ** 2, axis=-1) + gn
        what = jnp.exp(gn - jnp.max(gn))
        wn = what / jnp.sum(what)
        wn_d = jax.lax.stop_gradient(wn)
        ess = 1.0 / jnp.sum(wn_d ** 2)
        bins = jnp.cumsum(wn_d)
        bins = bins.at[-1].set(jnp.maximum(1.0, bins[-1]))
        idx = jnp.clip(jnp.digitize(dice_all[t], bins), 0, N_PART - 1)
        do = ess <= N_PART / 2.0
        x_new = jnp.where(do, x_end[idx], x_end)
        gn = jnp.where(do, jnp.zeros(N_PART, dtype=jnp.float32), gn)
        x_new = x_new.at[-1].set(input_path[ui])
        return (x_new.T, gn), (seg, idx, do, wn)

    (_, _), (segs, idx_all, do_all, wn_all) = jax.lax.scan(
        step, (x, gn), jnp.arange(T_STEPS))
    wn_final = wn_all[-1]

    dice1 = jax.random.uniform(jax.random.fold_in(key, 10 ** 6), (1,), dtype=jnp.float32)
    binsf = jnp.cumsum(jax.lax.stop_gradient(wn_final))
    binsf = binsf.at[-1].set(jnp.maximum(1.0, binsf[-1]))
    j = jnp.clip(jnp.digitize(dice1, binsf), 0, N_PART - 1)[0]

    def back(c, t):
        c_prev = jnp.where(do_all[t - 1], idx_all[t - 1, c], c)
        c_prev = jnp.where(c == N_PART - 1, N_PART - 1, c_prev)
        return c_prev, c

    c0, cs_rev = jax.lax.scan(back, j, jnp.arange(T_STEPS - 1, 0, -1))
    cs = jnp.concatenate([jnp.array([c0]), cs_rev[::-1]])

    def seg_rows(t):
        c = cs[t]
        own = segs[t, :, :, c]
        pin = jax.lax.dynamic_slice_in_dim(input_path, L * t + 1, L)
        return jnp.where(c == N_PART - 1, pin, own)

    rows = jax.vmap(seg_rows)(jnp.arange(T_STEPS)).reshape(T_STEPS * L, DX)
    row0 = jnp.where(cs[0] == N_PART - 1, input_path[0],
                     jnp.zeros(DX, jnp.float32))
    return jnp.concatenate([row0[None], rows], axis=0)


# speedup vs baseline: 15.1539x; 4.0277x over previous
import jax
import jax.numpy as jnp
from jax.experimental import pallas as pl

L_EXP = 5
T_STEPS = 20
N_PART = 2048
DX = 8
DY = 8
INIT_VAL = 0.0
L_SEG = 2 ** L_EXP


def _sim_segment_kernel(x_ref, sdw_ref, theta_ref, seg_ref):
    hl = jnp.float32(2.0 ** (-L_EXP))
    theta = theta_ref[...]

    def body(l, x):
        xn = x + (-theta * x) * hl + sdw_ref[l]
        seg_ref[l] = xn
        return xn

    jax.lax.fori_loop(0, L_SEG, body, x_ref[...])


def _simulate_segment(x_t, sdw_t, theta_col):
    return pl.pallas_call(
        _sim_segment_kernel,
        out_shape=jax.ShapeDtypeStruct((L_SEG, DX, N_PART), jnp.float32),
    )(x_t, sdw_t, theta_col)


_KB = 512


def _resample_kernel(dice_ref, lo_ref, hi_ref, xaug_ref, out_ref):
    dice = dice_ref[...]
    acc = jnp.zeros((N_PART, 16), jnp.float32)
    for kb in range(N_PART // _KB):
        sl = pl.ds(kb * _KB, _KB)
        lo = lo_ref[:, sl]
        hi = hi_ref[:, sl]
        onehot = jnp.where((dice >= lo) & (dice < hi),
                           jnp.float32(1.0), jnp.float32(0.0))
        acc += jnp.dot(onehot, xaug_ref[sl, :],
                       precision=jax.lax.Precision.HIGHEST,
                       preferred_element_type=jnp.float32)
    out_ref[...] = acc


def _resample(dice, bins, x_end):
    neg_inf = jnp.full((1,), -jnp.inf, dtype=jnp.float32)
    lo = jnp.concatenate([neg_inf, bins[:-1]])[None, :]
    hi = bins[None, :]
    slot = jax.lax.iota(jnp.float32, N_PART)[:, None]
    xaug = jnp.concatenate(
        [x_end, slot, jnp.zeros((N_PART, 16 - DX - 1), jnp.float32)], axis=1)
    out = pl.pallas_call(
        _resample_kernel,
        out_shape=jax.ShapeDtypeStruct((N_PART, 16), jnp.float32),
    )(dice[:, None], lo, hi, xaug)
    return out[:, :DX], out[:, DX].astype(jnp.int32)


def kernel(input_path, observe_path, theta, sigma_mat):
    key = jax.random.key(42)
    L = L_SEG
    hl = jnp.float32(2.0 ** (-L_EXP))
    theta_col = theta.reshape(DX, 1)

    step_keys = jax.vmap(lambda t: jax.random.fold_in(key, 2 * t))(jnp.arange(T_STEPS))
    sub_keys = jax.vmap(lambda k: jax.random.split(k, L))(step_keys)
    flat_keys = sub_keys.reshape(T_STEPS * L)

    dw_all = jax.vmap(
        lambda k: jax.random.normal(k, (N_PART, DX, 1), dtype=jnp.float32)
    )(flat_keys) * jnp.sqrt(hl)
    sdw_all = (sigma_mat @ dw_all)[..., 0].reshape(T_STEPS, L, N_PART, DX)
    sdw_all_t = sdw_all.transpose(0, 1, 3, 2)

    dice_all = jax.vmap(
        lambda t: jax.random.uniform(jax.random.fold_in(key, 2 * t + 1),
                                     (N_PART,), dtype=jnp.float32)
    )(jnp.arange(T_STEPS))

    x = jnp.full((DX, N_PART), INIT_VAL, dtype=jnp.float32)
    gn = jnp.zeros(N_PART, dtype=jnp.float32)

    def step(carry, t):
        x, gn = carry
        seg = _simulate_segment(x, sdw_all_t[t], theta_col)
        ui = L * (t + 1)
        x_end = seg[L - 1].T
        x_end = x_end.at[-1].set(input_path[ui])
        gn = -0.5 * jnp.sum((observe_path[t + 1] - x_end) ** 2, axis=-1) + gn
        what = jnp.exp(gn - jnp.max(gn))
        wn = what / jnp.sum(what)
        wn_d = jax.lax.stop_gradient(wn)
        ess = 1.0 / jnp.sum(wn_d ** 2)
        bins = jnp.cumsum(wn_d)
        bins = bins.at[-1].set(jnp.maximum(1.0, bins[-1]))
        x_res, idx = _resample(dice_all[t], bins, x_end)
        do = ess <= N_PART / 2.0
        x_new = jnp.where(do, x_res, x_end)
        gn = jnp.where(do, jnp.zeros(N_PART, dtype=jnp.float32), gn)
        x_new = x_new.at[-1].set(input_path[ui])
        return (x_new.T, gn), (seg, idx, do, wn)

    (_, _), (segs, idx_all, do_all, wn_all) = jax.lax.scan(
        step, (x, gn), jnp.arange(T_STEPS))
    wn_final = wn_all[-1]

    dice1 = jax.random.uniform(jax.random.fold_in(key, 10 ** 6), (1,), dtype=jnp.float32)
    binsf = jnp.cumsum(jax.lax.stop_gradient(wn_final))
    binsf = binsf.at[-1].set(jnp.maximum(1.0, binsf[-1]))
    j = jnp.clip(jnp.sum((binsf <= dice1[0]).astype(jnp.int32)), 0, N_PART - 1)

    def back(c, t):
        anc = jax.lax.dynamic_slice(idx_all, (t - 1, c), (1, 1))[0, 0]
        c_prev = jnp.where(do_all[t - 1], anc, c)
        c_prev = jnp.where(c == N_PART - 1, N_PART - 1, c_prev)
        return c_prev, c

    c0, cs_rev = jax.lax.scan(back, j, jnp.arange(T_STEPS - 1, 0, -1))
    cs = jnp.concatenate([jnp.array([c0]), cs_rev[::-1]])

    def seg_rows(t):
        c = cs[t]
        own = segs[t, :, :, c]
        pin = jax.lax.dynamic_slice_in_dim(input_path, L * t + 1, L)
        return jnp.where(c == N_PART - 1, pin, own)

    rows = jax.vmap(seg_rows)(jnp.arange(T_STEPS)).reshape(T_STEPS * L, DX)
    row0 = jnp.where(cs[0] == N_PART - 1, input_path[0],
                     jnp.zeros(DX, jnp.float32))
    return jnp.concatenate([row0[None], rows], axis=0)


# speedup vs baseline: 19.7991x; 1.3065x over previous
import functools

import jax
import jax.numpy as jnp
from jax.experimental import pallas as pl
from jax.experimental.pallas import tpu as pltpu
from jax.experimental.pallas import tpu_sc as plsc

L_EXP = 5
T_STEPS = 20
N_PART = 2048
DX = 8
DY = 8
INIT_VAL = 0.0
L_SEG = 2 ** L_EXP


def _sim_segment_kernel(x_ref, sdw_ref, theta_ref, seg_ref):
    hl = jnp.float32(2.0 ** (-L_EXP))
    theta = theta_ref[...]

    def body(l, x):
        xn = x + (-theta * x) * hl + sdw_ref[l]
        seg_ref[l] = xn
        return xn

    jax.lax.fori_loop(0, L_SEG, body, x_ref[...])


def _simulate_segment(x_t, sdw_t, theta_col):
    return pl.pallas_call(
        _sim_segment_kernel,
        out_shape=jax.ShapeDtypeStruct((L_SEG, DX, N_PART), jnp.float32),
    )(x_t, sdw_t, theta_col)


_KB = 512


def _resample_kernel(dice_ref, lo_ref, hi_ref, xaug_ref, out_ref):
    dice = dice_ref[...]
    acc = jnp.zeros((N_PART, 16), jnp.float32)
    for kb in range(N_PART // _KB):
        sl = pl.ds(kb * _KB, _KB)
        lo = lo_ref[:, sl]
        hi = hi_ref[:, sl]
        onehot = jnp.where((dice >= lo) & (dice < hi),
                           jnp.float32(1.0), jnp.float32(0.0))
        acc += jnp.dot(onehot, xaug_ref[sl, :],
                       precision=jax.lax.Precision.HIGHEST,
                       preferred_element_type=jnp.float32)
    out_ref[...] = acc


def _resample_tc(dice, bins, x_end):
    neg_inf = jnp.full((1,), -jnp.inf, dtype=jnp.float32)
    lo = jnp.concatenate([neg_inf, bins[:-1]])[None, :]
    hi = bins[None, :]
    slot = jax.lax.iota(jnp.float32, N_PART)[:, None]
    xaug = jnp.concatenate(
        [x_end, slot, jnp.zeros((N_PART, 16 - DX - 1), jnp.float32)], axis=1)
    out = pl.pallas_call(
        _resample_kernel,
        out_shape=jax.ShapeDtypeStruct((N_PART, 16), jnp.float32),
    )(dice[:, None], lo, hi, xaug)
    return out[:, :DX], out[:, DX].astype(jnp.int32)


_SC_LANES = 16
_SC_WORKERS = 32
_SC_CHUNK = N_PART // _SC_WORKERS
_SC_SEARCH_STEPS = 12


def _resample_sc_body(dice_hbm, bins_hbm, xflat_hbm, outx_hbm, outidx_hbm,
                      bins_v, x_v, dice_v, idx_v, rows_v):
    wid = jax.lax.axis_index("s") * 2 + jax.lax.axis_index("c")
    base = wid * _SC_CHUNK
    pltpu.sync_copy(bins_hbm, bins_v)
    pltpu.sync_copy(xflat_hbm, x_v)
    pltpu.sync_copy(dice_hbm.at[pl.ds(base, _SC_CHUNK)], dice_v)
    for v in range(_SC_CHUNK // _SC_LANES):
        d = dice_v[pl.ds(v * _SC_LANES, _SC_LANES)]
        lo = jnp.zeros((_SC_LANES,), jnp.int32)
        hi = jnp.full((_SC_LANES,), N_PART, jnp.int32)
        for _ in range(_SC_SEARCH_STEPS):
            mid = jax.lax.shift_right_logical(lo + hi, 1)
            bm = plsc.load_gather(bins_v, [mid])
            pred = bm <= d
            lo = jnp.where(pred, mid + 1, lo)
            hi = jnp.where(pred, hi, mid)
        idx = jnp.minimum(lo, N_PART - 1)
        idx_v[pl.ds(v * _SC_LANES, _SC_LANES)] = idx
        src_base = jax.lax.shift_left(idx, 3)
        dst_base = jax.lax.shift_left(
            jax.lax.iota(jnp.int32, _SC_LANES) + v * _SC_LANES, 3)
        for col in range(DX):
            vals = plsc.load_gather(x_v, [src_base + col])
            plsc.store_scatter(rows_v, [dst_base + col], vals)
    pltpu.sync_copy(idx_v, outidx_hbm.at[pl.ds(base, _SC_CHUNK)])
    pltpu.sync_copy(rows_v, outx_hbm.at[pl.ds(base * DX, _SC_CHUNK * DX)])


def _resample(dice, bins, x_end):
    call = pl.kernel(
        _resample_sc_body,
        out_type=(jax.ShapeDtypeStruct((N_PART * DX,), jnp.float32),
                  jax.ShapeDtypeStruct((N_PART,), jnp.int32)),
        mesh=plsc.VectorSubcoreMesh(core_axis_name="c", subcore_axis_name="s"),
        compiler_params=pltpu.CompilerParams(needs_layout_passes=False),
        scratch_types=[
            pltpu.VMEM((N_PART,), jnp.float32),
            pltpu.VMEM((N_PART * DX,), jnp.float32),
            pltpu.VMEM((_SC_CHUNK,), jnp.float32),
            pltpu.VMEM((_SC_CHUNK,), jnp.int32),
            pltpu.VMEM((_SC_CHUNK * DX,), jnp.float32),
        ],
    )
    xf, idx = call(dice, bins, x_end.reshape(N_PART * DX))
    return xf.reshape(N_PART, DX), idx


def kernel(input_path, observe_path, theta, sigma_mat):
    key = jax.random.key(42)
    L = L_SEG
    hl = jnp.float32(2.0 ** (-L_EXP))
    theta_col = theta.reshape(DX, 1)

    step_keys = jax.vmap(lambda t: jax.random.fold_in(key, 2 * t))(jnp.arange(T_STEPS))
    sub_keys = jax.vmap(lambda k: jax.random.split(k, L))(step_keys)
    flat_keys = sub_keys.reshape(T_STEPS * L)

    dw_all = jax.vmap(
        lambda k: jax.random.normal(k, (N_PART, DX, 1), dtype=jnp.float32)
    )(flat_keys) * jnp.sqrt(hl)
    sdw_all_t = jnp.einsum('ik,bnko->bin', sigma_mat, dw_all,
                           ).reshape(T_STEPS, L, DX, N_PART)

    dice_all = jax.vmap(
        lambda t: jax.random.uniform(jax.random.fold_in(key, 2 * t + 1),
                                     (N_PART,), dtype=jnp.float32)
    )(jnp.arange(T_STEPS))

    x = jnp.full((DX, N_PART), INIT_VAL, dtype=jnp.float32)
    gn = jnp.zeros(N_PART, dtype=jnp.float32)

    def step(carry, t):
        x, gn = carry
        seg = _simulate_segment(x, sdw_all_t[t], theta_col)
        ui = L * (t + 1)
        x_end = seg[L - 1].T
        x_end = x_end.at[-1].set(input_path[ui])
        gn = -0.5 * jnp.sum((observe_path[t + 1] - x_end) ** 2, axis=-1) + gn
        what = jnp.exp(gn - jnp.max(gn))
        wn = what / jnp.sum(what)
        wn_d = jax.lax.stop_gradient(wn)
        ess = 1.0 / jnp.sum(wn_d ** 2)
        bins = jnp.cumsum(wn_d)
        bins = bins.at[-1].set(jnp.maximum(1.0, bins[-1]))
        x_res, idx = _resample(dice_all[t], bins, x_end)
        do = ess <= N_PART / 2.0
        x_new = jnp.where(do, x_res, x_end)
        gn = jnp.where(do, jnp.zeros(N_PART, dtype=jnp.float32), gn)
        x_new = x_new.at[-1].set(input_path[ui])
        return (x_new.T, gn), (seg, idx, do, wn)

    (_, _), (segs, idx_all, do_all, wn_all) = jax.lax.scan(
        step, (x, gn), jnp.arange(T_STEPS))
    wn_final = wn_all[-1]

    dice1 = jax.random.uniform(jax.random.fold_in(key, 10 ** 6), (1,), dtype=jnp.float32)
    binsf = jnp.cumsum(jax.lax.stop_gradient(wn_final))
    binsf = binsf.at[-1].set(jnp.maximum(1.0, binsf[-1]))
    j = jnp.clip(jnp.sum((binsf <= dice1[0]).astype(jnp.int32)), 0, N_PART - 1)

    def back(c, t):
        anc = jax.lax.dynamic_slice(idx_all, (t - 1, c), (1, 1))[0, 0]
        c_prev = jnp.where(do_all[t - 1], anc, c)
        c_prev = jnp.where(c == N_PART - 1, N_PART - 1, c_prev)
        return c_prev, c

    c0, cs_rev = jax.lax.scan(back, j, jnp.arange(T_STEPS - 1, 0, -1))
    cs = jnp.concatenate([jnp.array([c0]), cs_rev[::-1]])

    def seg_rows(t):
        c = cs[t]
        own = segs[t, :, :, c]
        pin = jax.lax.dynamic_slice_in_dim(input_path, L * t + 1, L)
        return jnp.where(c == N_PART - 1, pin, own)

    rows = jax.vmap(seg_rows)(jnp.arange(T_STEPS)).reshape(T_STEPS * L, DX)
    row0 = jnp.where(cs[0] == N_PART - 1, input_path[0],
                     jnp.zeros(DX, jnp.float32))
    return jnp.concatenate([row0[None], rows], axis=0)


# speedup vs baseline: 20.2215x; 1.0213x over previous
import functools

import jax
import jax.numpy as jnp
from jax.experimental import pallas as pl
from jax.experimental.pallas import tpu as pltpu
from jax.experimental.pallas import tpu_sc as plsc

L_EXP = 5
T_STEPS = 20
N_PART = 2048
DX = 8
DY = 8
INIT_VAL = 0.0
L_SEG = 2 ** L_EXP


def _sim_segment_kernel(t_ref, x_ref, sdw_ref, theta_ref, seg_ref):
    del t_ref
    hl = jnp.float32(2.0 ** (-L_EXP))
    theta = theta_ref[...]

    def body(l, x):
        xn = x + (-theta * x) * hl + sdw_ref[0, l]
        seg_ref[l] = xn
        return xn

    jax.lax.fori_loop(0, L_SEG, body, x_ref[...])


def _simulate_segment(x_t, sdw_all_t, t, theta_col):
    return pl.pallas_call(
        _sim_segment_kernel,
        grid_spec=pltpu.PrefetchScalarGridSpec(
            num_scalar_prefetch=1,
            grid=(1,),
            in_specs=[
                pl.BlockSpec((DX, N_PART), lambda i, t: (0, 0)),
                pl.BlockSpec((1, L_SEG, DX, N_PART),
                             lambda i, t: (t[0], 0, 0, 0)),
                pl.BlockSpec((DX, 1), lambda i, t: (0, 0)),
            ],
            out_specs=pl.BlockSpec((L_SEG, DX, N_PART), lambda i, t: (0, 0, 0)),
        ),
        out_shape=jax.ShapeDtypeStruct((L_SEG, DX, N_PART), jnp.float32),
    )(t[None], x_t, sdw_all_t, theta_col)


_KB = 512


def _resample_kernel(dice_ref, lo_ref, hi_ref, xaug_ref, out_ref):
    dice = dice_ref[...]
    acc = jnp.zeros((N_PART, 16), jnp.float32)
    for kb in range(N_PART // _KB):
        sl = pl.ds(kb * _KB, _KB)
        lo = lo_ref[:, sl]
        hi = hi_ref[:, sl]
        onehot = jnp.where((dice >= lo) & (dice < hi),
                           jnp.float32(1.0), jnp.float32(0.0))
        acc += jnp.dot(onehot, xaug_ref[sl, :],
                       precision=jax.lax.Precision.HIGHEST,
                       preferred_element_type=jnp.float32)
    out_ref[...] = acc


def _resample_tc(dice, bins, x_end):
    neg_inf = jnp.full((1,), -jnp.inf, dtype=jnp.float32)
    lo = jnp.concatenate([neg_inf, bins[:-1]])[None, :]
    hi = bins[None, :]
    slot = jax.lax.iota(jnp.float32, N_PART)[:, None]
    xaug = jnp.concatenate(
        [x_end, slot, jnp.zeros((N_PART, 16 - DX - 1), jnp.float32)], axis=1)
    out = pl.pallas_call(
        _resample_kernel,
        out_shape=jax.ShapeDtypeStruct((N_PART, 16), jnp.float32),
    )(dice[:, None], lo, hi, xaug)
    return out[:, :DX], out[:, DX].astype(jnp.int32)


_SC_LANES = 16
_SC_WORKERS = 32
_SC_CHUNK = N_PART // _SC_WORKERS
_SC_SEARCH_STEPS = 12


def _resample_sc_body(dice_hbm, bins_hbm, xflat_hbm, outx_hbm, outidx_hbm,
                      bins_v, x_v, dice_v, idx_v, rows_v):
    wid = jax.lax.axis_index("s") * 2 + jax.lax.axis_index("c")
    base = wid * _SC_CHUNK
    pltpu.sync_copy(bins_hbm, bins_v)
    pltpu.sync_copy(xflat_hbm, x_v)
    pltpu.sync_copy(dice_hbm.at[pl.ds(base, _SC_CHUNK)], dice_v)
    for v in range(_SC_CHUNK // _SC_LANES):
        d = dice_v[pl.ds(v * _SC_LANES, _SC_LANES)]
        lo = jnp.zeros((_SC_LANES,), jnp.int32)
        hi = jnp.full((_SC_LANES,), N_PART, jnp.int32)
        for _ in range(_SC_SEARCH_STEPS):
            mid = jax.lax.shift_right_logical(lo + hi, 1)
            bm = plsc.load_gather(bins_v, [mid])
            pred = bm <= d
            lo = jnp.where(pred, mid + 1, lo)
            hi = jnp.where(pred, hi, mid)
        idx = jnp.minimum(lo, N_PART - 1)
        idx_v[pl.ds(v * _SC_LANES, _SC_LANES)] = idx
        src_base = jax.lax.shift_left(idx, 3)
        dst_base = jax.lax.shift_left(
            jax.lax.iota(jnp.int32, _SC_LANES) + v * _SC_LANES, 3)
        for col in range(DX):
            vals = plsc.load_gather(x_v, [src_base + col])
            plsc.store_scatter(rows_v, [dst_base + col], vals)
    pltpu.sync_copy(idx_v, outidx_hbm.at[pl.ds(base, _SC_CHUNK)])
    pltpu.sync_copy(rows_v, outx_hbm.at[pl.ds(base * DX, _SC_CHUNK * DX)])


def _resample(dice, bins, x_end):
    call = pl.kernel(
        _resample_sc_body,
        out_type=(jax.ShapeDtypeStruct((N_PART * DX,), jnp.float32),
                  jax.ShapeDtypeStruct((N_PART,), jnp.int32)),
        mesh=plsc.VectorSubcoreMesh(core_axis_name="c", subcore_axis_name="s"),
        compiler_params=pltpu.CompilerParams(needs_layout_passes=False),
        scratch_types=[
            pltpu.VMEM((N_PART,), jnp.float32),
            pltpu.VMEM((N_PART * DX,), jnp.float32),
            pltpu.VMEM((_SC_CHUNK,), jnp.float32),
            pltpu.VMEM((_SC_CHUNK,), jnp.int32),
            pltpu.VMEM((_SC_CHUNK * DX,), jnp.float32),
        ],
    )
    xf, idx = call(dice, bins, x_end.reshape(N_PART * DX))
    return xf.reshape(N_PART, DX), idx


def kernel(input_path, observe_path, theta, sigma_mat):
    key = jax.random.key(42)
    L = L_SEG
    hl = jnp.float32(2.0 ** (-L_EXP))
    theta_col = theta.reshape(DX, 1)

    step_keys = jax.vmap(lambda t: jax.random.fold_in(key, 2 * t))(jnp.arange(T_STEPS))
    sub_keys = jax.vmap(lambda k: jax.random.split(k, L))(step_keys)
    flat_keys = sub_keys.reshape(T_STEPS * L)

    dw_all = jax.vmap(
        lambda k: jax.random.normal(k, (N_PART, DX, 1), dtype=jnp.float32)
    )(flat_keys) * jnp.sqrt(hl)
    sdw_all_t = jnp.einsum('ik,bnko->bin', sigma_mat, dw_all,
                           ).reshape(T_STEPS, L, DX, N_PART)

    dice_all = jax.vmap(
        lambda t: jax.random.uniform(jax.random.fold_in(key, 2 * t + 1),
                                     (N_PART,), dtype=jnp.float32)
    )(jnp.arange(T_STEPS))

    x = jnp.full((DX, N_PART), INIT_VAL, dtype=jnp.float32)
    gn = jnp.zeros(N_PART, dtype=jnp.float32)

    def step(carry, t):
        x, gn = carry
        seg = _simulate_segment(x, sdw_all_t, t, theta_col)
        ui = L * (t + 1)
        x_end = seg[L - 1].T
        x_end = x_end.at[-1].set(input_path[ui])
        gn = -0.5 * jnp.sum((observe_path[t + 1] - x_end) ** 2, axis=-1) + gn
        what = jnp.exp(gn - jnp.max(gn))
        wn = what / jnp.sum(what)
        wn_d = jax.lax.stop_gradient(wn)
        ess = 1.0 / jnp.sum(wn_d ** 2)
        bins = jnp.cumsum(wn_d)
        bins = bins.at[-1].set(jnp.maximum(1.0, bins[-1]))
        x_res, idx = _resample(dice_all[t], bins, x_end)
        do = ess <= N_PART / 2.0
        x_new = jnp.where(do, x_res, x_end)
        gn = jnp.where(do, jnp.zeros(N_PART, dtype=jnp.float32), gn)
        x_new = x_new.at[-1].set(input_path[ui])
        return (x_new.T, gn), (seg, idx, do, wn)

    (_, _), (segs, idx_all, do_all, wn_all) = jax.lax.scan(
        step, (x, gn), jnp.arange(T_STEPS))
    wn_final = wn_all[-1]

    dice1 = jax.random.uniform(jax.random.fold_in(key, 10 ** 6), (1,), dtype=jnp.float32)
    binsf = jnp.cumsum(jax.lax.stop_gradient(wn_final))
    binsf = binsf.at[-1].set(jnp.maximum(1.0, binsf[-1]))
    j = jnp.clip(jnp.sum((binsf <= dice1[0]).astype(jnp.int32)), 0, N_PART - 1)

    def back(c, t):
        anc = jax.lax.dynamic_slice(idx_all, (t - 1, c), (1, 1))[0, 0]
        c_prev = jnp.where(do_all[t - 1], anc, c)
        c_prev = jnp.where(c == N_PART - 1, N_PART - 1, c_prev)
        return c_prev, c

    c0, cs_rev = jax.lax.scan(back, j, jnp.arange(T_STEPS - 1, 0, -1))
    cs = jnp.concatenate([jnp.array([c0]), cs_rev[::-1]])

    def seg_rows(t):
        c = cs[t]
        own = segs[t, :, :, c]
        pin = jax.lax.dynamic_slice_in_dim(input_path, L * t + 1, L)
        return jnp.where(c == N_PART - 1, pin, own)

    rows = jax.vmap(seg_rows)(jnp.arange(T_STEPS)).reshape(T_STEPS * L, DX)
    row0 = jnp.where(cs[0] == N_PART - 1, input_path[0],
                     jnp.zeros(DX, jnp.float32))
    return jnp.concatenate([row0[None], rows], axis=0)


# speedup vs baseline: 20.6124x; 1.0193x over previous
import functools

import jax
import jax.numpy as jnp
from jax.experimental import pallas as pl
from jax.experimental.pallas import tpu as pltpu
from jax.experimental.pallas import tpu_sc as plsc

L_EXP = 5
T_STEPS = 20
N_PART = 2048
DX = 8
DY = 8
INIT_VAL = 0.0
L_SEG = 2 ** L_EXP


def _sim_segment_kernel(t_ref, x_ref, sdw_ref, theta_ref, seg_ref):
    del t_ref
    hl = jnp.float32(2.0 ** (-L_EXP))
    theta = theta_ref[...]

    def body(l, x):
        xn = x + (-theta * x) * hl + sdw_ref[0, l]
        seg_ref[l] = xn
        return xn

    jax.lax.fori_loop(0, L_SEG, body, x_ref[...])


def _simulate_segment(x_t, sdw_all_t, t, theta_col):
    return pl.pallas_call(
        _sim_segment_kernel,
        grid_spec=pltpu.PrefetchScalarGridSpec(
            num_scalar_prefetch=1,
            grid=(1,),
            in_specs=[
                pl.BlockSpec((DX, N_PART), lambda i, t: (0, 0)),
                pl.BlockSpec((1, L_SEG, DX, N_PART),
                             lambda i, t: (t[0], 0, 0, 0)),
                pl.BlockSpec((DX, 1), lambda i, t: (0, 0)),
            ],
            out_specs=pl.BlockSpec((L_SEG, DX, N_PART), lambda i, t: (0, 0, 0)),
        ),
        out_shape=jax.ShapeDtypeStruct((L_SEG, DX, N_PART), jnp.float32),
    )(t[None], x_t, sdw_all_t, theta_col)


_KB = 512


def _resample_kernel(dice_ref, lo_ref, hi_ref, xaug_ref, out_ref):
    dice = dice_ref[...]
    acc = jnp.zeros((N_PART, 16), jnp.float32)
    for kb in range(N_PART // _KB):
        sl = pl.ds(kb * _KB, _KB)
        lo = lo_ref[:, sl]
        hi = hi_ref[:, sl]
        onehot = jnp.where((dice >= lo) & (dice < hi),
                           jnp.float32(1.0), jnp.float32(0.0))
        acc += jnp.dot(onehot, xaug_ref[sl, :],
                       precision=jax.lax.Precision.HIGHEST,
                       preferred_element_type=jnp.float32)
    out_ref[...] = acc


def _resample_tc(dice, bins, x_end):
    neg_inf = jnp.full((1,), -jnp.inf, dtype=jnp.float32)
    lo = jnp.concatenate([neg_inf, bins[:-1]])[None, :]
    hi = bins[None, :]
    slot = jax.lax.iota(jnp.float32, N_PART)[:, None]
    xaug = jnp.concatenate(
        [x_end, slot, jnp.zeros((N_PART, 16 - DX - 1), jnp.float32)], axis=1)
    out = pl.pallas_call(
        _resample_kernel,
        out_shape=jax.ShapeDtypeStruct((N_PART, 16), jnp.float32),
    )(dice[:, None], lo, hi, xaug)
    return out[:, :DX], out[:, DX].astype(jnp.int32)


_SC_LANES = 16
_SC_WORKERS = 32
_SC_CHUNK = N_PART // _SC_WORKERS
_SC_SEARCH_STEPS = 12


def _resample_sc_body(dice_hbm, bins_hbm, xflat_hbm, outx_hbm, outidx_hbm,
                      bins_v, x_v, dice_v, idx_v, rows_v):
    wid = jax.lax.axis_index("s") * 2 + jax.lax.axis_index("c")
    base = wid * _SC_CHUNK
    pltpu.sync_copy(bins_hbm, bins_v)
    pltpu.sync_copy(xflat_hbm, x_v)
    pltpu.sync_copy(dice_hbm.at[pl.ds(base, _SC_CHUNK)], dice_v)
    for v in range(_SC_CHUNK // _SC_LANES):
        d = dice_v[pl.ds(v * _SC_LANES, _SC_LANES)]
        lo = jnp.zeros((_SC_LANES,), jnp.int32)
        hi = jnp.full((_SC_LANES,), N_PART, jnp.int32)
        for _ in range(_SC_SEARCH_STEPS):
            mid = jax.lax.shift_right_logical(lo + hi, 1)
            bm = plsc.load_gather(bins_v, [mid])
            pred = bm <= d
            lo = jnp.where(pred, mid + 1, lo)
            hi = jnp.where(pred, hi, mid)
        idx = jnp.minimum(lo, N_PART - 1)
        idx_v[pl.ds(v * _SC_LANES, _SC_LANES)] = idx
        src_base = jax.lax.shift_left(idx, 3)
        dst_base = jax.lax.shift_left(
            jax.lax.iota(jnp.int32, _SC_LANES) + v * _SC_LANES, 3)
        for col in range(DX):
            vals = plsc.load_gather(x_v, [src_base + col])
            plsc.store_scatter(rows_v, [dst_base + col], vals)
    pltpu.sync_copy(idx_v, outidx_hbm.at[pl.ds(base, _SC_CHUNK)])
    pltpu.sync_copy(rows_v, outx_hbm.at[pl.ds(base * DX, _SC_CHUNK * DX)])


def _resample(dice, bins, x_end):
    call = pl.kernel(
        _resample_sc_body,
        out_type=(jax.ShapeDtypeStruct((N_PART * DX,), jnp.float32),
                  jax.ShapeDtypeStruct((N_PART,), jnp.int32)),
        mesh=plsc.VectorSubcoreMesh(core_axis_name="c", subcore_axis_name="s"),
        compiler_params=pltpu.CompilerParams(needs_layout_passes=False),
        scratch_types=[
            pltpu.VMEM((N_PART,), jnp.float32),
            pltpu.VMEM((N_PART * DX,), jnp.float32),
            pltpu.VMEM((_SC_CHUNK,), jnp.float32),
            pltpu.VMEM((_SC_CHUNK,), jnp.int32),
            pltpu.VMEM((_SC_CHUNK * DX,), jnp.float32),
        ],
    )
    xf, idx = call(dice, bins, x_end.reshape(N_PART * DX))
    return xf.reshape(N_PART, DX), idx


def kernel(input_path, observe_path, theta, sigma_mat):
    key = jax.random.key(42)
    L = L_SEG
    hl = jnp.float32(2.0 ** (-L_EXP))
    theta_col = theta.reshape(DX, 1)

    step_keys = jax.vmap(lambda t: jax.random.fold_in(key, 2 * t))(jnp.arange(T_STEPS))
    sub_keys = jax.vmap(lambda k: jax.random.split(k, L))(step_keys)
    flat_keys = sub_keys.reshape(T_STEPS * L)

    dw_all = jax.vmap(
        lambda k: jax.random.normal(k, (N_PART, DX, 1), dtype=jnp.float32)
    )(flat_keys) * jnp.sqrt(hl)
    sdw_all_t = jnp.einsum('ik,bnko->bin', sigma_mat, dw_all,
                           ).reshape(T_STEPS, L, DX, N_PART)

    dice_all = jax.vmap(
        lambda t: jax.random.uniform(jax.random.fold_in(key, 2 * t + 1),
                                     (N_PART,), dtype=jnp.float32)
    )(jnp.arange(T_STEPS))

    x = jnp.full((DX, N_PART), INIT_VAL, dtype=jnp.float32)
    gn = jnp.zeros(N_PART, dtype=jnp.float32)

    def step(carry, t):
        x, gn = carry
        seg = _simulate_segment(x, sdw_all_t, t, theta_col)
        ui = L * (t + 1)
        x_end = seg[L - 1].T
        x_end = x_end.at[-1].set(input_path[ui])
        gn = -0.5 * jnp.sum((observe_path[t + 1] - x_end) ** 2, axis=-1) + gn
        what = jnp.exp(gn - jnp.max(gn))
        wn = what / jnp.sum(what)
        wn_d = jax.lax.stop_gradient(wn)
        ess = 1.0 / jnp.sum(wn_d ** 2)
        bins = jnp.cumsum(wn_d)
        bins = bins.at[-1].set(jnp.maximum(1.0, bins[-1]))
        x_res, idx = _resample(dice_all[t], bins, x_end)
        do = ess <= N_PART / 2.0
        x_new = jnp.where(do, x_res, x_end)
        gn = jnp.where(do, jnp.zeros(N_PART, dtype=jnp.float32), gn)
        x_new = x_new.at[-1].set(input_path[ui])
        return (x_new.T, gn), (seg, idx, do, wn)

    (_, _), (segs, idx_all, do_all, wn_all) = jax.lax.scan(
        step, (x, gn), jnp.arange(T_STEPS))
    wn_final = wn_all[-1]

    dice1 = jax.random.uniform(jax.random.fold_in(key, 10 ** 6), (1,), dtype=jnp.float32)
    binsf = jnp.cumsum(jax.lax.stop_gradient(wn_final))
    binsf = binsf.at[-1].set(jnp.maximum(1.0, binsf[-1]))
    j = jnp.clip(jnp.sum((binsf <= dice1[0]).astype(jnp.int32)), 0, N_PART - 1)

    slots = jnp.arange(N_PART, dtype=idx_all.dtype)[None, :]
    eff_idx = jnp.where(do_all[:, None], idx_all, slots)
    eff_idx = eff_idx.at[:, -1].set(N_PART - 1)

    def back(c, t):
        c_prev = jax.lax.dynamic_slice(eff_idx, (t - 1, c), (1, 1))[0, 0]
        return c_prev, c

    c0, cs_rev = jax.lax.scan(back, j, jnp.arange(T_STEPS - 1, 0, -1))
    cs = jnp.concatenate([jnp.array([c0]), cs_rev[::-1]])

    def seg_rows(t):
        c = cs[t]
        own = segs[t, :, :, c]
        pin = jax.lax.dynamic_slice_in_dim(input_path, L * t + 1, L)
        return jnp.where(c == N_PART - 1, pin, own)

    rows = jax.vmap(seg_rows)(jnp.arange(T_STEPS)).reshape(T_STEPS * L, DX)
    row0 = jnp.where(cs[0] == N_PART - 1, input_path[0],
                     jnp.zeros(DX, jnp.float32))
    return jnp.concatenate([row0[None], rows], axis=0)


# speedup vs baseline: 20.6195x; 1.0003x over previous
import jax
import jax.numpy as jnp
from jax.experimental import pallas as pl
from jax.experimental.pallas import tpu as pltpu
from jax.experimental.pallas import tpu_sc as plsc

L_EXP = 5
T_STEPS = 20
N_PART = 2048
DX = 8
DY = 8
INIT_VAL = 0.0
L_SEG = 2 ** L_EXP


def _sim_segment_kernel(t_ref, x_ref, sdw_ref, theta_ref, seg_ref):
    del t_ref
    hl = jnp.float32(2.0 ** (-L_EXP))
    theta = theta_ref[...]

    def body(l, x):
        xn = x + (-theta * x) * hl + sdw_ref[0, l]
        seg_ref[l] = xn
        return xn

    jax.lax.fori_loop(0, L_SEG, body, x_ref[...])


def _simulate_segment(x_t, sdw_all_t, t, theta_col):
    return pl.pallas_call(
        _sim_segment_kernel,
        grid_spec=pltpu.PrefetchScalarGridSpec(
            num_scalar_prefetch=1,
            grid=(1,),
            in_specs=[
                pl.BlockSpec((DX, N_PART), lambda i, t: (0, 0)),
                pl.BlockSpec((1, L_SEG, DX, N_PART),
                             lambda i, t: (t[0], 0, 0, 0)),
                pl.BlockSpec((DX, 1), lambda i, t: (0, 0)),
            ],
            out_specs=pl.BlockSpec((L_SEG, DX, N_PART), lambda i, t: (0, 0, 0)),
        ),
        out_shape=jax.ShapeDtypeStruct((L_SEG, DX, N_PART), jnp.float32),
    )(t[None], x_t, sdw_all_t, theta_col)


_SC_LANES = 16
_SC_WORKERS = 32
_SC_CHUNK = N_PART // _SC_WORKERS
_SC_SEARCH_STEPS = 12


def _resample_sc_body(dice_hbm, bins_hbm, xflat_hbm, outx_hbm, outidx_hbm,
                      bins_v, x_v, dice_v, idx_v, rows_v):
    wid = jax.lax.axis_index("s") * 2 + jax.lax.axis_index("c")
    base = wid * _SC_CHUNK
    pltpu.sync_copy(bins_hbm, bins_v)
    pltpu.sync_copy(xflat_hbm, x_v)
    pltpu.sync_copy(dice_hbm.at[pl.ds(base, _SC_CHUNK)], dice_v)
    for v in range(_SC_CHUNK // _SC_LANES):
        d = dice_v[pl.ds(v * _SC_LANES, _SC_LANES)]
        lo = jnp.zeros((_SC_LANES,), jnp.int32)
        hi = jnp.full((_SC_LANES,), N_PART, jnp.int32)
        for _ in range(_SC_SEARCH_STEPS):
            mid = jax.lax.shift_right_logical(lo + hi, 1)
            bm = plsc.load_gather(bins_v, [mid])
            pred = bm <= d
            lo = jnp.where(pred, mid + 1, lo)
            hi = jnp.where(pred, hi, mid)
        idx = jnp.minimum(lo, N_PART - 1)
        idx_v[pl.ds(v * _SC_LANES, _SC_LANES)] = idx
        src_base = jax.lax.shift_left(idx, 3)
        dst_base = jax.lax.shift_left(
            jax.lax.iota(jnp.int32, _SC_LANES) + v * _SC_LANES, 3)
        for col in range(DX):
            vals = plsc.load_gather(x_v, [src_base + col])
            plsc.store_scatter(rows_v, [dst_base + col], vals)
    pltpu.sync_copy(idx_v, outidx_hbm.at[pl.ds(base, _SC_CHUNK)])
    pltpu.sync_copy(rows_v, outx_hbm.at[pl.ds(base * DX, _SC_CHUNK * DX)])


def _resample(dice, bins, x_end):
    call = pl.kernel(
        _resample_sc_body,
        out_type=(jax.ShapeDtypeStruct((N_PART * DX,), jnp.float32),
                  jax.ShapeDtypeStruct((N_PART,), jnp.int32)),
        mesh=plsc.VectorSubcoreMesh(core_axis_name="c", subcore_axis_name="s"),
        compiler_params=pltpu.CompilerParams(needs_layout_passes=False),
        scratch_types=[
            pltpu.VMEM((N_PART,), jnp.float32),
            pltpu.VMEM((N_PART * DX,), jnp.float32),
            pltpu.VMEM((_SC_CHUNK,), jnp.float32),
            pltpu.VMEM((_SC_CHUNK,), jnp.int32),
            pltpu.VMEM((_SC_CHUNK * DX,), jnp.float32),
        ],
    )
    xf, idx = call(dice, bins, x_end.reshape(N_PART * DX))
    return xf.reshape(N_PART, DX), idx


def kernel(input_path, observe_path, theta, sigma_mat):
    key = jax.random.key(42)
    L = L_SEG
    hl = jnp.float32(2.0 ** (-L_EXP))
    theta_col = theta.reshape(DX, 1)

    step_keys = jax.vmap(lambda t: jax.random.fold_in(key, 2 * t))(jnp.arange(T_STEPS))
    sub_keys = jax.vmap(lambda k: jax.random.split(k, L))(step_keys)
    flat_keys = sub_keys.reshape(T_STEPS * L)

    dw_all = jax.vmap(
        lambda k: jax.random.normal(k, (N_PART, DX, 1), dtype=jnp.float32)
    )(flat_keys) * jnp.sqrt(hl)
    sdw_all_t = jnp.einsum('ik,bnko->bin', sigma_mat, dw_all,
                           ).reshape(T_STEPS, L, DX, N_PART)

    dice_all = jax.vmap(
        lambda t: jax.random.uniform(jax.random.fold_in(key, 2 * t + 1),
                                     (N_PART,), dtype=jnp.float32)
    )(jnp.arange(T_STEPS))

    x = jnp.full((DX, N_PART), INIT_VAL, dtype=jnp.float32)
    gn = jnp.zeros(N_PART, dtype=jnp.float32)

    def step(carry, t):
        x, gn = carry
        seg = _simulate_segment(x, sdw_all_t, t, theta_col)
        ui = L * (t + 1)
        x_end = seg[L - 1].T
        x_end = x_end.at[-1].set(input_path[ui])
        gn = -0.5 * jnp.sum((observe_path[t + 1] - x_end) ** 2, axis=-1) + gn
        what = jnp.exp(gn - jnp.max(gn))
        wn = what / jnp.sum(what)
        wn_d = jax.lax.stop_gradient(wn)
        ess = 1.0 / jnp.sum(wn_d ** 2)
        bins = jnp.cumsum(wn_d)
        bins = bins.at[-1].set(jnp.maximum(1.0, bins[-1]))
        x_res, idx = _resample(dice_all[t], bins, x_end)
        do = ess <= N_PART / 2.0
        x_new = jnp.where(do, x_res, x_end)
        gn = jnp.where(do, jnp.zeros(N_PART, dtype=jnp.float32), gn)
        x_new = x_new.at[-1].set(input_path[ui])
        return (x_new.T, gn), (seg, idx, do, wn)

    (_, _), (segs, idx_all, do_all, wn_all) = jax.lax.scan(
        step, (x, gn), jnp.arange(T_STEPS))
    wn_final = wn_all[-1]

    dice1 = jax.random.uniform(jax.random.fold_in(key, 10 ** 6), (1,), dtype=jnp.float32)
    binsf = jnp.cumsum(jax.lax.stop_gradient(wn_final))
    binsf = binsf.at[-1].set(jnp.maximum(1.0, binsf[-1]))
    j = jnp.clip(jnp.sum((binsf <= dice1[0]).astype(jnp.int32)), 0, N_PART - 1)

    slots = jnp.arange(N_PART, dtype=idx_all.dtype)[None, :]
    eff_idx = jnp.where(do_all[:, None], idx_all, slots)
    eff_idx = eff_idx.at[:, -1].set(N_PART - 1)

    def back(c, t):
        c_prev = jax.lax.dynamic_slice(eff_idx, (t - 1, c), (1, 1))[0, 0]
        return c_prev, c

    c0, cs_rev = jax.lax.scan(back, j, jnp.arange(T_STEPS - 1, 0, -1))
    cs = jnp.concatenate([jnp.array([c0]), cs_rev[::-1]])

    def seg_rows(t):
        c = cs[t]
        own = segs[t, :, :, c]
        pin = jax.lax.dynamic_slice_in_dim(input_path, L * t + 1, L)
        return jnp.where(c == N_PART - 1, pin, own)

    rows = jax.vmap(seg_rows)(jnp.arange(T_STEPS)).reshape(T_STEPS * L, DX)
    row0 = jnp.where(cs[0] == N_PART - 1, input_path[0],
                     jnp.zeros(DX, jnp.float32))
    return jnp.concatenate([row0[None], rows], axis=0)
